# baseline (speedup 1.0000x reference)
import jax
import jax.numpy as jnp
from jax.experimental import pallas as pl
from jax.experimental.pallas import tpu as pltpu

N = 100000
D = 128
B = 256
STEPS = 3
R = 4000
NB = N // R
W = 32

_NEG = -1e30


def _set2set_kernel(x_ref, b_ref, lo_ref, nw_ref,
                    wih_ref, whh_ref, bih_ref, bhh_ref,
                    out_ref, h_scr, c_scr, qs_scr,
                    m_scr, d_scr, u_scr, mf_scr, df_scr, uf_scr,
                    mb_scr, db_scr, ub_scr,
                    xh_scr, st0, st1, sem0, sem1):
    f32 = jnp.float32
    bf16 = jnp.bfloat16

    stages = (st0, st1)
    sems = (sem0, sem1)

    def _copy(blk):
        return pltpu.make_async_copy(
            x_ref.at[pl.ds(blk * R, R), :], stages[blk % 2], sems[blk % 2])

    _copy(0).start()
    for blk in range(NB):
        if blk + 1 < NB:
            _copy(blk + 1).start()
        _copy(blk).wait()
        xh_scr[pl.ds(blk * R, R), :] = stages[blk % 2][...].astype(bf16)

    h_scr[...] = jnp.zeros((B, D), f32)
    c_scr[...] = jnp.zeros((B, D), f32)
    qs_scr[...] = jnp.zeros((B, 2 * D), f32)

    for _ in range(STEPS):
        qs = qs_scr[...]
        h = h_scr[...]
        c = c_scr[...]
        gates = (jax.lax.dot_general(qs, wih_ref[...],
                                     (((1,), (1,)), ((), ())),
                                     preferred_element_type=f32)
                 + jax.lax.dot_general(h, whh_ref[...],
                                       (((1,), (1,)), ((), ())),
                                       preferred_element_type=f32)
                 + bih_ref[...] + bhh_ref[...])
        ig = jax.nn.sigmoid(gates[:, 0 * D:1 * D])
        fg = jax.nn.sigmoid(gates[:, 1 * D:2 * D])
        gg = jnp.tanh(gates[:, 2 * D:3 * D])
        og = jax.nn.sigmoid(gates[:, 3 * D:4 * D])
        c = fg * c + ig * gg
        h = og * jnp.tanh(c)
        h_scr[...] = h
        c_scr[...] = c

        m_scr[...] = jnp.full((B, 128), _NEG, f32)
        d_scr[...] = jnp.zeros((B, 128), f32)
        mf_scr[...] = jnp.full((B, 128), _NEG, f32)
        df_scr[...] = jnp.zeros((B, 128), f32)
        uf_scr[...] = jnp.zeros((B, D), f32)

        def blk_core(blk, lo, w):
            xbh = xh_scr[pl.ds(blk * R, R), :]
            bb = b_ref[pl.ds(blk, 1), :]
            oh = (bb - lo) == jax.lax.broadcasted_iota(
                jnp.int32, (w, 1), 0)
            ohf = oh.astype(f32)
            qw = h_scr[pl.ds(lo, w), :].astype(jnp.bfloat16)
            E = jax.lax.dot_general(qw, xbh, (((1,), (1,)), ((), ())),
                                    preferred_element_type=f32)
            Em = jnp.where(oh, E, _NEG)
            bm = jnp.max(Em, axis=1, keepdims=True)
            cen = jnp.sum(ohf * (E - bm), axis=0, keepdims=True)
            ex = jnp.exp(cen)
            Wm = ohf * ex
            dd = jnp.sum(Wm, axis=1, keepdims=True)
            Ub = jax.lax.dot_general(Wm.astype(jnp.bfloat16), xbh,
                                     (((1,), (0,)), ((), ())),
                                     preferred_element_type=f32)
            return bm, dd, Ub

        def sweep(blk, _):
            @pl.when(nw_ref[blk] == 1)
            def _narrow():
                bm, dd, Ub = blk_core(blk, lo_ref[blk], W)
                mb_scr[blk] = bm
                db_scr[blk] = dd
                ub_scr[blk] = Ub

            @pl.when(nw_ref[blk] == 0)
            def _full():
                bm, dd, Ub = blk_core(blk, 0, B)
                m_old = mf_scr[:, 0:1]
                m_new = jnp.maximum(m_old, bm)
                sc_old = jnp.exp(m_old - m_new)
                sc_new = jnp.exp(bm - m_new)
                df_scr[:, 0:1] = df_scr[:, 0:1] * sc_old + dd * sc_new
                uf_scr[...] = uf_scr[...] * sc_old + Ub * sc_new
                mf_scr[:, 0:1] = m_new
            return 0

        jax.lax.fori_loop(0, NB, sweep, 0)

        def comb_max(blk, _):
            @pl.when(nw_ref[blk] == 1)
            def _():
                lo = lo_ref[blk]
                m_scr[pl.ds(lo, W), 0:1] = jnp.maximum(
                    m_scr[pl.ds(lo, W), 0:1], mb_scr[blk])
            return 0

        jax.lax.fori_loop(0, NB, comb_max, 0)

        u_scr[...] = jnp.zeros((B, D), f32)

        def comb_add(blk, _):
            @pl.when(nw_ref[blk] == 1)
            def _():
                lo = lo_ref[blk]
                f = jnp.exp(mb_scr[blk] - m_scr[pl.ds(lo, W), 0:1])
                d_scr[pl.ds(lo, W), 0:1] += db_scr[blk] * f
                u_scr[pl.ds(lo, W), :] += ub_scr[blk] * f
            return 0

        jax.lax.fori_loop(0, NB, comb_add, 0)

        m_n = m_scr[:, 0:1]
        m_f = mf_scr[:, 0:1]
        m_t = jnp.maximum(m_n, m_f)
        f_n = jnp.exp(m_n - m_t)
        f_f = jnp.exp(m_f - m_t)
        d_t = d_scr[:, 0:1] * f_n + df_scr[:, 0:1] * f_f
        u_t = u_scr[...] * f_n + uf_scr[...] * f_f
        qs_scr[:, 0:D] = h
        qs_scr[:, D:2 * D] = u_t / (d_t + 1e-16)

    out_ref[...] = qs_scr[...]


@jax.jit
def kernel(x, batch, W_ih, W_hh, b_ih, b_hh):
    batch2d = batch.astype(jnp.int32).reshape(NB, R)
    first = batch2d[:, 0]
    last = batch2d[:, -1]
    lo = jnp.minimum(jnp.bitwise_and(first, -8), B - W)
    narrow = (last - lo < W).astype(jnp.int32)
    bih2d = b_ih.reshape(1, 4 * D)
    bhh2d = b_hh.reshape(1, 4 * D)
    out = pl.pallas_call(
        _set2set_kernel,
        in_specs=[
            pl.BlockSpec(memory_space=pl.ANY),
            pl.BlockSpec(memory_space=pltpu.VMEM),
            pl.BlockSpec(memory_space=pltpu.SMEM),
            pl.BlockSpec(memory_space=pltpu.SMEM),
            pl.BlockSpec(memory_space=pltpu.VMEM),
            pl.BlockSpec(memory_space=pltpu.VMEM),
            pl.BlockSpec(memory_space=pltpu.VMEM),
            pl.BlockSpec(memory_space=pltpu.VMEM),
        ],
        out_specs=pl.BlockSpec(memory_space=pltpu.VMEM),
        out_shape=jax.ShapeDtypeStruct((B, 2 * D), jnp.float32),
        scratch_shapes=[
            pltpu.VMEM((B, D), jnp.float32),
            pltpu.VMEM((B, D), jnp.float32),
            pltpu.VMEM((B, 2 * D), jnp.float32),
            pltpu.VMEM((B, 128), jnp.float32),
            pltpu.VMEM((B, 128), jnp.float32),
            pltpu.VMEM((B, D), jnp.float32),
            pltpu.VMEM((B, 128), jnp.float32),
            pltpu.VMEM((B, 128), jnp.float32),
            pltpu.VMEM((B, D), jnp.float32),
            pltpu.VMEM((NB, W, 1), jnp.float32),
            pltpu.VMEM((NB, W, 1), jnp.float32),
            pltpu.VMEM((NB, W, D), jnp.float32),
            pltpu.VMEM((N, D), jnp.bfloat16),
            pltpu.VMEM((R, D), jnp.float32),
            pltpu.VMEM((R, D), jnp.float32),
            pltpu.SemaphoreType.DMA,
            pltpu.SemaphoreType.DMA,
        ],
        compiler_params=pltpu.CompilerParams(
            vmem_limit_bytes=100 * 1024 * 1024,
        ),
    )(x, batch2d, lo, narrow, W_ih, W_hh, bih2d, bhh2d)
    return out


# speedup vs baseline: 1.1368x; 1.1368x over previous
import jax
import jax.numpy as jnp
from jax.experimental import pallas as pl
from jax.experimental.pallas import tpu as pltpu

N = 100000
D = 128
B = 256
STEPS = 3
R = 4000
NB = N // R
W = 32

_NEG = -1e30


def _set2set_kernel(x_ref, b_ref, lo_ref, nw_ref,
                    wih_ref, whh_ref, bih_ref, bhh_ref,
                    out_ref, h_scr, c_scr, qs_scr,
                    m_scr, d_scr, u_scr, mf_scr, df_scr, uf_scr,
                    mb_scr, db_scr, ub_scr):
    f32 = jnp.float32

    h_scr[...] = jnp.zeros((B, D), f32)
    c_scr[...] = jnp.zeros((B, D), f32)
    qs_scr[...] = jnp.zeros((B, 2 * D), f32)

    for _ in range(STEPS):
        qs = qs_scr[...]
        h = h_scr[...]
        c = c_scr[...]
        gates = (jax.lax.dot_general(qs, wih_ref[...],
                                     (((1,), (1,)), ((), ())),
                                     preferred_element_type=f32)
                 + jax.lax.dot_general(h, whh_ref[...],
                                       (((1,), (1,)), ((), ())),
                                       preferred_element_type=f32)
                 + bih_ref[...] + bhh_ref[...])
        ig = jax.nn.sigmoid(gates[:, 0 * D:1 * D])
        fg = jax.nn.sigmoid(gates[:, 1 * D:2 * D])
        gg = jnp.tanh(gates[:, 2 * D:3 * D])
        og = jax.nn.sigmoid(gates[:, 3 * D:4 * D])
        c = fg * c + ig * gg
        h = og * jnp.tanh(c)
        h_scr[...] = h
        c_scr[...] = c

        m_scr[...] = jnp.full((B, 128), _NEG, f32)
        d_scr[...] = jnp.zeros((B, 128), f32)
        mf_scr[...] = jnp.full((B, 128), _NEG, f32)
        df_scr[...] = jnp.zeros((B, 128), f32)
        uf_scr[...] = jnp.zeros((B, D), f32)

        def blk_core(blk, lo, w):
            xbh = x_ref[pl.ds(blk * R, R), :]
            bb = b_ref[pl.ds(blk, 1), :]
            oh = (bb - lo) == jax.lax.broadcasted_iota(
                jnp.int32, (w, 1), 0)
            ohf = oh.astype(f32)
            qw = h_scr[pl.ds(lo, w), :]
            E = jax.lax.dot_general(qw, xbh, (((1,), (1,)), ((), ())),
                                    preferred_element_type=f32)
            Em = jnp.where(oh, E, _NEG)
            bm = jnp.max(Em, axis=1, keepdims=True)
            cen = jnp.sum(ohf * (E - bm), axis=0, keepdims=True)
            ex = jnp.exp(cen)
            Wm = ohf * ex
            dd = jnp.sum(Wm, axis=1, keepdims=True)
            Ub = jax.lax.dot_general(Wm, xbh, (((1,), (0,)), ((), ())),
                                     preferred_element_type=f32)
            return bm, dd, Ub

        def sweep(blk, _):
            @pl.when(nw_ref[blk] == 1)
            def _narrow():
                bm, dd, Ub = blk_core(blk, lo_ref[blk], W)
                mb_scr[blk] = bm
                db_scr[blk] = dd
                ub_scr[blk] = Ub

            @pl.when(nw_ref[blk] == 0)
            def _full():
                bm, dd, Ub = blk_core(blk, 0, B)
                m_old = mf_scr[:, 0:1]
                m_new = jnp.maximum(m_old, bm)
                sc_old = jnp.exp(m_old - m_new)
                sc_new = jnp.exp(bm - m_new)
                df_scr[:, 0:1] = df_scr[:, 0:1] * sc_old + dd * sc_new
                uf_scr[...] = uf_scr[...] * sc_old + Ub * sc_new
                mf_scr[:, 0:1] = m_new
            return 0

        jax.lax.fori_loop(0, NB, sweep, 0)

        for blk in range(NB):
            @pl.when(nw_ref[blk] == 1)
            def _(blk=blk):
                lo = lo_ref[blk]
                m_scr[pl.ds(lo, W), 0:1] = jnp.maximum(
                    m_scr[pl.ds(lo, W), 0:1], mb_scr[blk])

        u_scr[...] = jnp.zeros((B, D), f32)

        for blk in range(NB):
            @pl.when(nw_ref[blk] == 1)
            def _(blk=blk):
                lo = lo_ref[blk]
                f = jnp.exp(mb_scr[blk] - m_scr[pl.ds(lo, W), 0:1])
                d_scr[pl.ds(lo, W), 0:1] += db_scr[blk] * f
                u_scr[pl.ds(lo, W), :] += ub_scr[blk] * f

        m_n = m_scr[:, 0:1]
        m_f = mf_scr[:, 0:1]
        m_t = jnp.maximum(m_n, m_f)
        f_n = jnp.exp(m_n - m_t)
        f_f = jnp.exp(m_f - m_t)
        d_t = d_scr[:, 0:1] * f_n + df_scr[:, 0:1] * f_f
        u_t = u_scr[...] * f_n + uf_scr[...] * f_f
        qs_scr[:, 0:D] = h
        qs_scr[:, D:2 * D] = u_t / (d_t + 1e-16)

    out_ref[...] = qs_scr[...]


@jax.jit
def kernel(x, batch, W_ih, W_hh, b_ih, b_hh):
    batch2d = batch.astype(jnp.int32).reshape(NB, R)
    first = batch2d[:, 0]
    last = batch2d[:, -1]
    lo = jnp.minimum(jnp.bitwise_and(first, -8), B - W)
    narrow = (last - lo < W).astype(jnp.int32)
    bih2d = b_ih.reshape(1, 4 * D)
    bhh2d = b_hh.reshape(1, 4 * D)
    out = pl.pallas_call(
        _set2set_kernel,
        in_specs=[
            pl.BlockSpec(memory_space=pltpu.VMEM),
            pl.BlockSpec(memory_space=pltpu.VMEM),
            pl.BlockSpec(memory_space=pltpu.SMEM),
            pl.BlockSpec(memory_space=pltpu.SMEM),
            pl.BlockSpec(memory_space=pltpu.VMEM),
            pl.BlockSpec(memory_space=pltpu.VMEM),
            pl.BlockSpec(memory_space=pltpu.VMEM),
            pl.BlockSpec(memory_space=pltpu.VMEM),
        ],
        out_specs=pl.BlockSpec(memory_space=pltpu.VMEM),
        out_shape=jax.ShapeDtypeStruct((B, 2 * D), jnp.float32),
        scratch_shapes=[
            pltpu.VMEM((B, D), jnp.float32),
            pltpu.VMEM((B, D), jnp.float32),
            pltpu.VMEM((B, 2 * D), jnp.float32),
            pltpu.VMEM((B, 128), jnp.float32),
            pltpu.VMEM((B, 128), jnp.float32),
            pltpu.VMEM((B, D), jnp.float32),
            pltpu.VMEM((B, 128), jnp.float32),
            pltpu.VMEM((B, 128), jnp.float32),
            pltpu.VMEM((B, D), jnp.float32),
            pltpu.VMEM((NB, W, 1), jnp.float32),
            pltpu.VMEM((NB, W, 1), jnp.float32),
            pltpu.VMEM((NB, W, D), jnp.float32),
        ],
        compiler_params=pltpu.CompilerParams(
            vmem_limit_bytes=100 * 1024 * 1024,
        ),
    )(x, batch2d, lo, narrow, W_ih, W_hh, bih2d, bhh2d)
    return out
